# Initial kernel scaffold; baseline (speedup 1.0000x reference)
import functools

import jax
import jax.numpy as jnp
from jax import lax
from jax.experimental import pallas as pl
from jax.experimental.pallas import tpu as pltpu
from jax.experimental.pallas import tpu_sc as plsc

N = 10000
E = 320000
D = 128
R = 8
NPAD = 10240
NC = 2
NS = 16
NW = NC * NS
CH = 128
NCHUNK = E // CH
CNT_W = 8
CNT_ROWS = 82048
CNT_TILE = CNT_ROWS // NS
DUMP_KEY = R * NPAD
EC = 327680
CROWS = EC // CH

_mesh = plsc.VectorSubcoreMesh(core_axis_name="c", subcore_axis_name="s")


@functools.partial(
    pl.kernel,
    out_type=jax.ShapeDtypeStruct((NC, CNT_ROWS, CNT_W), jnp.float32),
    mesh=_mesh,
    scratch_types=[
        pltpu.VMEM((8, CH), jnp.int32),
        pltpu.VMEM((CH, CNT_W), jnp.float32),
        pltpu.VMEM_SHARED((CNT_ROWS, CNT_W), jnp.float32),
    ],
)
def _counts_kernel(key2p_h, ones_h, zeros_h, cnt_h, kidx, onesb, cntsp):
    cid = lax.axis_index("c")
    sid = lax.axis_index("s")
    pltpu.sync_copy(ones_h, onesb)
    pltpu.sync_copy(zeros_h, cntsp.at[pl.ds(sid * CNT_TILE, CNT_TILE)])
    plsc.subcore_barrier()
    base = cid * (CROWS // NC) + sid * (CROWS // NC // NS)

    def grp(gi, carry):
        row = base + gi * 8
        pltpu.sync_copy(key2p_h.at[pl.ds(row, 8)], kidx)
        for j in range(8):
            pltpu.sync_copy(onesb, cntsp.at[kidx.at[j]], add=True)
        return carry

    lax.fori_loop(0, CROWS // NC // NS // 8, grp, 0)
    plsc.subcore_barrier()
    pltpu.sync_copy(cntsp.at[pl.ds(sid * CNT_TILE, CNT_TILE)],
                    cnt_h.at[cid, pl.ds(sid * CNT_TILE, CNT_TILE)])


@functools.partial(
    pl.kernel,
    out_type=(jax.ShapeDtypeStruct((E,), jnp.float32),
              jax.ShapeDtypeStruct((NPAD, D), jnp.float32)),
    mesh=_mesh,
    scratch_types=[
        pltpu.VMEM((CH,), jnp.int32),
        pltpu.VMEM((CH, CNT_W), jnp.float32),
        pltpu.VMEM((CH, CNT_W), jnp.float32),
        pltpu.VMEM((CH,), jnp.float32),
        pltpu.VMEM((64,), jnp.int32),
        pltpu.VMEM((64, D), jnp.float32),
        pltpu.SemaphoreType.DMA,
    ],
)
def _prep_kernel(cnt0_h, cnt1_h, key2_h, xp_h, embed_h, w_h, h0_h,
                 kb, cr0, cr1, wb, xib, hrows, sem):
    cid = lax.axis_index("c")
    sid = lax.axis_index("s")
    gw = sid * NC + cid
    n = 78 + (gw < 4).astype(jnp.int32)
    base = gw * 78 + jnp.minimum(gw, 4)

    def chunk(ci, carry):
        ch = base + ci
        off = pl.multiple_of(ch * CH, CH)
        pltpu.sync_copy(key2_h.at[pl.ds(off, CH)], kb)
        pltpu.async_copy(cnt0_h.at[kb], cr0, sem).wait()
        pltpu.async_copy(cnt1_h.at[kb], cr1, sem).wait()
        z16 = jnp.zeros((16,), jnp.int32)
        for j in range(8):
            idx = lax.iota(jnp.int32, 16) + j * 16
            c0 = plsc.load_gather(cr0, [idx, z16])
            c1 = plsc.load_gather(cr1, [idx, z16])
            wb[pl.ds(j * 16, 16)] = 1.0 / (c0 + c1)
        pltpu.sync_copy(wb, w_h.at[pl.ds(off, CH)])
        return carry

    lax.fori_loop(0, n, chunk, 0)
    for c in range(NPAD // NW // 64):
        o = gw * (NPAD // NW) + c * 64
        pltpu.sync_copy(xp_h.at[pl.ds(o, 64)], xib)
        pltpu.async_copy(embed_h.at[xib], hrows, sem).wait()
        pltpu.sync_copy(hrows, h0_h.at[pl.ds(o, 64)])


@functools.partial(
    pl.kernel,
    out_type=jax.ShapeDtypeStruct((NC, NPAD, D), jnp.float32),
    mesh=_mesh,
    scratch_types=[
        pltpu.VMEM((CH,), jnp.int32),
        pltpu.VMEM((1, CH), jnp.int32),
        pltpu.VMEM((CH,), jnp.float32),
        pltpu.VMEM((CH, D), jnp.float32),
        pltpu.VMEM_SHARED((NPAD, D), jnp.float32),
        pltpu.SemaphoreType.DMA,
    ],
)
def _edge_kernel(zz_h, g_h, dst2d_h, w_h, zeros_h, msg_h,
                 gb, db, wb, rows, acc, sem):
    cid = lax.axis_index("c")
    sid = lax.axis_index("s")
    gw = sid * NC + cid
    for c in range(NPAD // NS // CH):
        pltpu.sync_copy(zeros_h, acc.at[pl.ds(sid * (NPAD // NS) + c * CH, CH)])
    plsc.subcore_barrier()
    n = 78 + (gw < 4).astype(jnp.int32)
    base = gw * 78 + jnp.minimum(gw, 4)

    def chunk(ci, carry):
        ch = base + ci
        off = pl.multiple_of(ch * CH, CH)
        pltpu.sync_copy(g_h.at[pl.ds(off, CH)], gb)
        pltpu.sync_copy(w_h.at[pl.ds(off, CH)], wb)
        pltpu.sync_copy(dst2d_h.at[pl.ds(ch, 1)], db)
        pltpu.async_copy(zz_h.at[gb], rows, sem).wait()

        def scale(k, c2):
            kv = jnp.full((16,), k, jnp.int32)
            wv = plsc.load_gather(wb, [kv])
            for j in range(D // 16):
                rows[k, pl.ds(j * 16, 16)] = rows[k, pl.ds(j * 16, 16)] * wv
            return c2

        lax.fori_loop(0, CH, scale, 0)
        pltpu.sync_copy(rows, acc.at[db.at[0]], add=True)
        return carry

    lax.fori_loop(0, n, chunk, 0)
    plsc.subcore_barrier()
    for c in range(NPAD // NS // CH):
        s = sid * (NPAD // NS) + c * CH
        pltpu.sync_copy(acc.at[pl.ds(s, CH)], msg_h.at[cid, pl.ds(s, CH)])


def _mm_body(h_ref, w_ref, o_ref):
    o_ref[0] = jnp.dot(h_ref[...], w_ref[0],
                       preferred_element_type=jnp.float32)


def _matmul(h, wall):
    BM = 512
    return pl.pallas_call(
        _mm_body,
        grid=(R + 1, NPAD // BM),
        in_specs=[pl.BlockSpec((BM, D), lambda r, i: (i, 0)),
                  pl.BlockSpec((1, D, D), lambda r, i: (r, 0, 0))],
        out_specs=pl.BlockSpec((1, BM, D), lambda r, i: (r, i, 0)),
        out_shape=jax.ShapeDtypeStruct((R + 1, NPAD, D), jnp.float32),
    )(h, wall)


def _combine(zz, msg, bias, do_relu):
    BM = 512

    def body(z_ref, m_ref, b_ref, o_ref):
        s = z_ref[0] + m_ref[0] + m_ref[1] + b_ref[...]
        if do_relu:
            s = jnp.maximum(s, 0.0)
        o_ref[...] = s

    return pl.pallas_call(
        body,
        grid=(NPAD // BM,),
        in_specs=[pl.BlockSpec((1, BM, D), lambda i: (0, i, 0)),
                  pl.BlockSpec((NC, BM, D), lambda i: (0, i, 0)),
                  pl.BlockSpec((1, D), lambda i: (0, 0))],
        out_specs=pl.BlockSpec((BM, D), lambda i: (i, 0)),
        out_shape=jax.ShapeDtypeStruct((NPAD, D), jnp.float32),
    )(zz, msg, bias.reshape(1, D))


def kernel(x, edge_index, edge_type, embed_weight, W1, root1, b1,
           W2, root2, b2):
    src = edge_index[0].astype(jnp.int32)
    dst = edge_index[1].astype(jnp.int32)
    et = edge_type.astype(jnp.int32)
    key2 = et * NPAD + dst
    key2p = jnp.pad(key2, (0, EC - E),
                    constant_values=DUMP_KEY).reshape(CROWS, CH)
    g = (et + 1) * NPAD + src
    dst2d = dst.reshape(NCHUNK, CH)
    xp = jnp.pad(x.astype(jnp.int32), (0, NPAD - N))
    ones_in = jnp.ones((CH, CNT_W), jnp.float32)
    zeros_cnt = jnp.zeros((CNT_TILE, CNT_W), jnp.float32)
    zeros_acc = jnp.zeros((CH, D), jnp.float32)

    cnt = _counts_kernel(key2p, ones_in, zeros_cnt)
    w, h0 = _prep_kernel(cnt[0], cnt[1], key2, xp, embed_weight)

    wall1 = jnp.concatenate([root1[None], W1], axis=0)
    wall2 = jnp.concatenate([root2[None], W2], axis=0)

    zz1 = _matmul(h0, wall1)
    msg1 = _edge_kernel(zz1.reshape((R + 1) * NPAD, D), g, dst2d, w, zeros_acc)
    h1 = _combine(zz1, msg1, b1, True)

    zz2 = _matmul(h1, wall2)
    msg2 = _edge_kernel(zz2.reshape((R + 1) * NPAD, D), g, dst2d, w, zeros_acc)
    out = _combine(zz2, msg2, b2, False)
    return out[:N]


# speedup vs baseline: 10.6518x; 10.6518x over previous
import functools

import jax
import jax.numpy as jnp
from jax import lax
from jax.experimental import pallas as pl
from jax.experimental.pallas import tpu as pltpu
from jax.experimental.pallas import tpu_sc as plsc

N = 10000
E = 320000
D = 128
R = 8
NPAD = 10240
NC = 2
NS = 16
NW = NC * NS
CH = 128
NCHUNK = E // CH
CNT_W = 16
CNT_ROWS = 82048
CNT_TILE = CNT_ROWS // NS
DUMP_KEY = R * NPAD
EC = 327680
CROWS = EC // CH

_mesh = plsc.VectorSubcoreMesh(core_axis_name="c", subcore_axis_name="s")
_sc_params = pltpu.CompilerParams(use_tc_tiling_on_sc=False)


@functools.partial(
    pl.kernel,
    out_type=jax.ShapeDtypeStruct((NC, CNT_ROWS, CNT_W), jnp.float32),
    mesh=_mesh,
    compiler_params=_sc_params,
    scratch_types=[
        pltpu.VMEM((8, CH), jnp.int32),
        pltpu.VMEM((CH, CNT_W), jnp.float32),
        pltpu.VMEM_SHARED((CNT_ROWS, CNT_W), jnp.float32),
    ],
)
def _counts_kernel(key2p_h, ones_h, zeros_h, cnt_h, kidx, onesb, cntsp):
    cid = lax.axis_index("c")
    sid = lax.axis_index("s")
    pltpu.sync_copy(ones_h, onesb)
    pltpu.sync_copy(zeros_h, cntsp.at[pl.ds(sid * CNT_TILE, CNT_TILE)])
    plsc.subcore_barrier()
    base = cid * (CROWS // NC) + sid * (CROWS // NC // NS)

    def grp(gi, carry):
        row = base + gi * 8
        pltpu.sync_copy(key2p_h.at[pl.ds(row, 8)], kidx)
        for j in range(8):
            pltpu.sync_copy(onesb, cntsp.at[kidx.at[j]], add=True)
        return carry

    lax.fori_loop(0, CROWS // NC // NS // 8, grp, 0)
    plsc.subcore_barrier()
    pltpu.sync_copy(cntsp.at[pl.ds(sid * CNT_TILE, CNT_TILE)],
                    cnt_h.at[cid, pl.ds(sid * CNT_TILE, CNT_TILE)])


@functools.partial(
    pl.kernel,
    out_type=(jax.ShapeDtypeStruct((E, CNT_W), jnp.float32),
              jax.ShapeDtypeStruct((NPAD, D), jnp.float32)),
    mesh=_mesh,
    compiler_params=_sc_params,
    scratch_types=[
        pltpu.VMEM((CH,), jnp.int32),
        pltpu.VMEM((CH, CNT_W), jnp.float32),
        pltpu.VMEM((CH, CNT_W), jnp.float32),
        pltpu.VMEM((CH, CNT_W), jnp.float32),
        pltpu.VMEM((64,), jnp.int32),
        pltpu.VMEM((64, D), jnp.float32),
        pltpu.SemaphoreType.DMA,
    ],
)
def _prep_kernel(cnt0_h, cnt1_h, key2_h, xp_h, embed_h, w_h, h0_h,
                 kb, cr0, cr1, wb, xib, hrows, sem):
    cid = lax.axis_index("c")
    sid = lax.axis_index("s")
    gw = sid * NC + cid
    n = 78 + (gw < 4).astype(jnp.int32)
    base = gw * 78 + jnp.minimum(gw, 4)

    def chunk(ci, carry):
        ch = base + ci
        off = pl.multiple_of(ch * CH, CH)
        pltpu.sync_copy(key2_h.at[pl.ds(off, CH)], kb)
        pltpu.async_copy(cnt0_h.at[kb], cr0, sem).wait()
        pltpu.async_copy(cnt1_h.at[kb], cr1, sem).wait()

        def wrow(i, c3):
            wb[i, pl.ds(0, CNT_W)] = 1.0 / (cr0[i, pl.ds(0, CNT_W)]
                                            + cr1[i, pl.ds(0, CNT_W)])
            return c3

        lax.fori_loop(0, CH, wrow, 0)
        pltpu.sync_copy(wb, w_h.at[pl.ds(off, CH)])
        return carry

    lax.fori_loop(0, n, chunk, 0)
    for c in range(NPAD // NW // 64):
        o = gw * (NPAD // NW) + c * 64
        pltpu.sync_copy(xp_h.at[pl.ds(o, 64)], xib)
        pltpu.async_copy(embed_h.at[xib], hrows, sem).wait()
        pltpu.sync_copy(hrows, h0_h.at[pl.ds(o, 64)])


@functools.partial(
    pl.kernel,
    out_type=jax.ShapeDtypeStruct((NC, NPAD, D), jnp.float32),
    mesh=_mesh,
    compiler_params=_sc_params,
    scratch_types=[
        pltpu.VMEM((CH,), jnp.int32),
        pltpu.VMEM((1, CH), jnp.int32),
        pltpu.VMEM((CH, CNT_W), jnp.float32),
        pltpu.VMEM((CH, D), jnp.float32),
        pltpu.VMEM_SHARED((NPAD, D), jnp.float32),
        pltpu.SemaphoreType.DMA,
    ],
)
def _edge_kernel(zz_h, g_h, dst2d_h, w_h, zeros_h, msg_h,
                 gb, db, wb, rows, acc, sem):
    cid = lax.axis_index("c")
    sid = lax.axis_index("s")
    gw = sid * NC + cid
    for c in range(NPAD // NS // CH):
        pltpu.sync_copy(zeros_h, acc.at[pl.ds(sid * (NPAD // NS) + c * CH, CH)])
    plsc.subcore_barrier()
    n = 78 + (gw < 4).astype(jnp.int32)
    base = gw * 78 + jnp.minimum(gw, 4)

    def chunk(ci, carry):
        ch = base + ci
        off = pl.multiple_of(ch * CH, CH)
        pltpu.sync_copy(g_h.at[pl.ds(off, CH)], gb)
        pltpu.sync_copy(w_h.at[pl.ds(off, CH)], wb)
        pltpu.sync_copy(dst2d_h.at[pl.ds(ch, 1)], db)
        pltpu.async_copy(zz_h.at[gb], rows, sem).wait()

        def scale(k, c2):
            wv = wb[k, pl.ds(0, CNT_W)]
            for j in range(D // 16):
                rows[k, pl.ds(j * 16, 16)] = rows[k, pl.ds(j * 16, 16)] * wv
            return c2

        lax.fori_loop(0, CH, scale, 0)
        pltpu.sync_copy(rows, acc.at[db.at[0]], add=True)
        return carry

    lax.fori_loop(0, n, chunk, 0)
    plsc.subcore_barrier()
    for c in range(NPAD // NS // CH):
        s = sid * (NPAD // NS) + c * CH
        pltpu.sync_copy(acc.at[pl.ds(s, CH)], msg_h.at[cid, pl.ds(s, CH)])


def _mm_body(h_ref, w_ref, o_ref):
    o_ref[0] = jnp.dot(h_ref[...], w_ref[0],
                       preferred_element_type=jnp.float32)


def _matmul(h, wall):
    BM = 512
    return pl.pallas_call(
        _mm_body,
        grid=(R + 1, NPAD // BM),
        in_specs=[pl.BlockSpec((BM, D), lambda r, i: (i, 0)),
                  pl.BlockSpec((1, D, D), lambda r, i: (r, 0, 0))],
        out_specs=pl.BlockSpec((1, BM, D), lambda r, i: (r, i, 0)),
        out_shape=jax.ShapeDtypeStruct((R + 1, NPAD, D), jnp.float32),
    )(h, wall)


def _combine(zz, msg, bias, do_relu):
    BM = 512

    def body(z_ref, m_ref, b_ref, o_ref):
        s = z_ref[0] + m_ref[0] + m_ref[1] + b_ref[...]
        if do_relu:
            s = jnp.maximum(s, 0.0)
        o_ref[...] = s

    return pl.pallas_call(
        body,
        grid=(NPAD // BM,),
        in_specs=[pl.BlockSpec((1, BM, D), lambda i: (0, i, 0)),
                  pl.BlockSpec((NC, BM, D), lambda i: (0, i, 0)),
                  pl.BlockSpec((1, D), lambda i: (0, 0))],
        out_specs=pl.BlockSpec((BM, D), lambda i: (i, 0)),
        out_shape=jax.ShapeDtypeStruct((NPAD, D), jnp.float32),
    )(zz, msg, bias.reshape(1, D))


def kernel(x, edge_index, edge_type, embed_weight, W1, root1, b1,
           W2, root2, b2):
    src = edge_index[0].astype(jnp.int32)
    dst = edge_index[1].astype(jnp.int32)
    et = edge_type.astype(jnp.int32)
    key2 = et * NPAD + dst
    key2p = jnp.pad(key2, (0, EC - E),
                    constant_values=DUMP_KEY).reshape(CROWS, CH)
    g = (et + 1) * NPAD + src
    dst2d = dst.reshape(NCHUNK, CH)
    xp = jnp.pad(x.astype(jnp.int32), (0, NPAD - N))
    ones_in = jnp.ones((CH, CNT_W), jnp.float32)
    zeros_cnt = jnp.zeros((CNT_TILE, CNT_W), jnp.float32)
    zeros_acc = jnp.zeros((CH, D), jnp.float32)

    cnt = _counts_kernel(key2p, ones_in, zeros_cnt)
    w, h0 = _prep_kernel(cnt[0], cnt[1], key2, xp, embed_weight)

    wall1 = jnp.concatenate([root1[None], W1], axis=0)
    wall2 = jnp.concatenate([root2[None], W2], axis=0)

    zz1 = _matmul(h0, wall1)
    msg1 = _edge_kernel(zz1.reshape((R + 1) * NPAD, D), g, dst2d, w, zeros_acc)
    h1 = _combine(zz1, msg1, b1, True)

    zz2 = _matmul(h1, wall2)
    msg2 = _edge_kernel(zz2.reshape((R + 1) * NPAD, D), g, dst2d, w, zeros_acc)
    out = _combine(zz2, msg2, b2, False)
    return out[:N]
